# baseline (speedup 1.0000x reference)
import functools

import jax
import jax.numpy as jnp
from jax import lax
from jax.experimental import pallas as pl
from jax.experimental.pallas import tpu as pltpu
from jax.experimental.pallas import tpu_sc as plsc

N_NODES = 100000
EMB_DIM = 64
D_FEAT = 128
D_OUT = EMB_DIM + D_FEAT

CH = 2000
NUM_CHUNKS = N_NODES // CH
NUM_WORKERS = 32
MAX_CHUNKS_PER_WORKER = -(-NUM_CHUNKS // NUM_WORKERS)

_MESH = plsc.VectorSubcoreMesh(core_axis_name="c", subcore_axis_name="s")


@functools.partial(
    pl.kernel,
    out_type=jax.ShapeDtypeStruct((N_NODES, D_OUT), jnp.float32),
    mesh=_MESH,
    scratch_types=[
        pltpu.VMEM((CH,), jnp.int32),
        pltpu.VMEM((CH, EMB_DIM), jnp.float32),
        pltpu.SemaphoreType.DMA,
    ],
    compiler_params=pltpu.CompilerParams(use_tc_tiling_on_sc=False),
)
def _feature_prep_sc(ids_hbm, feats_hbm, table_hbm, out_hbm, idx_v, rows_v, sem):
    wid = lax.axis_index("s") * _MESH.num_cores + lax.axis_index("c")
    for i in range(MAX_CHUNKS_PER_WORKER):
        chunk = wid + i * NUM_WORKERS

        @pl.when(chunk < NUM_CHUNKS)
        def _():
            base = chunk * CH
            pltpu.sync_copy(ids_hbm.at[pl.ds(base, CH)], idx_v)
            gather = pltpu.async_copy(table_hbm.at[idx_v], rows_v, sem)
            pltpu.sync_copy(
                feats_hbm.at[pl.ds(base, CH)],
                out_hbm.at[pl.ds(base, CH), pl.ds(EMB_DIM, D_FEAT)],
            )
            gather.wait()
            pltpu.sync_copy(
                rows_v, out_hbm.at[pl.ds(base, CH), pl.ds(0, EMB_DIM)]
            )


def kernel(ids, feats, table):
    return _feature_prep_sc(ids.astype(jnp.int32), feats, table)


# speedup vs baseline: 2.7230x; 2.7230x over previous
import functools

import jax
import jax.numpy as jnp
from jax import lax
from jax.experimental import pallas as pl
from jax.experimental.pallas import tpu as pltpu
from jax.experimental.pallas import tpu_sc as plsc

N_NODES = 100000
EMB_DIM = 64
D_FEAT = 128
D_OUT = EMB_DIM + D_FEAT

CH = 160
NUM_CHUNKS = N_NODES // CH
NUM_WORKERS = 32
MAX_CHUNKS_PER_WORKER = -(-NUM_CHUNKS // NUM_WORKERS)

_MESH = plsc.VectorSubcoreMesh(core_axis_name="c", subcore_axis_name="s")


@functools.partial(
    pl.kernel,
    out_type=jax.ShapeDtypeStruct((N_NODES, D_OUT), jnp.float32),
    mesh=_MESH,
    scratch_types=[
        pltpu.VMEM((CH + 16,), jnp.int32),
        pltpu.VMEM((CH,), jnp.int32),
        pltpu.VMEM((CH, 2 * EMB_DIM), jnp.float32),
        pltpu.VMEM((CH, D_FEAT), jnp.float32),
        pltpu.VMEM((CH, D_OUT), jnp.float32),
        pltpu.SemaphoreType.DMA,
    ],
)
def _feature_prep_sc(
    ids_hbm, feats_hbm, table2_hbm, out_hbm,
    idx_v, idx2_v, pairs_v, feats_v, row_v, sem,
):
    wid = lax.axis_index("s") * _MESH.num_cores + lax.axis_index("c")
    for i in range(MAX_CHUNKS_PER_WORKER):
        chunk = wid + i * NUM_WORKERS

        @pl.when(chunk < NUM_CHUNKS)
        def _():
            base = chunk * CH
            pltpu.sync_copy(ids_hbm.at[pl.ds(base, CH)], idx_v.at[pl.ds(0, CH)])
            for q in range(CH // 16):
                idx2_v[pl.ds(q * 16, 16)] = (
                    idx_v[pl.ds(q * 16, 16)] >> 1
                )
            gather = pltpu.async_copy(table2_hbm.at[idx2_v], pairs_v, sem)
            pltpu.sync_copy(feats_hbm.at[pl.ds(base, CH)], feats_v)
            gather.wait()

            def assemble(r, carry):
                sel = (idx_v[pl.ds(r, 16)][0] & 1) * EMB_DIM
                for q in range(EMB_DIM // 16):
                    row_v[r, pl.ds(q * 16, 16)] = pairs_v[
                        r, pl.ds(sel + q * 16, 16)
                    ]
                for q in range(D_FEAT // 16):
                    row_v[r, pl.ds(EMB_DIM + q * 16, 16)] = feats_v[
                        r, pl.ds(q * 16, 16)
                    ]
                return carry

            lax.fori_loop(0, CH, assemble, 0)
            pltpu.sync_copy(row_v, out_hbm.at[pl.ds(base, CH)])


def kernel(ids, feats, table):
    table2 = table.reshape(table.shape[0] // 2, 2 * table.shape[1])
    return _feature_prep_sc(ids.astype(jnp.int32), feats, table2)
